# baseline (speedup 1.0000x reference)
import jax
import jax.numpy as jnp
from jax import lax
from jax.experimental import pallas as pl
from jax.experimental.pallas import tpu as pltpu


def _addend_body(months_ref, ch_ref, pos_ref, mon_ref, out_ref):
    b, rows, d = out_ref.shape
    bs, n = ch_ref.shape
    t = months_ref.shape[1]
    reps = rows // (t * bs)
    mv = months_ref[...]
    ch = ch_ref[...]
    pe = pos_ref[...]
    mt = mon_ref[...]
    zero = jnp.zeros((bs, n), jnp.float32)
    iota_m = lax.broadcasted_iota(jnp.int32, (t, mt.shape[0]), 1)
    for bi in range(b):
        onehot = (mv[bi][:, None] == iota_m).astype(jnp.float32)
        me_b = jax.lax.dot_general(
            onehot, mt, dimension_numbers=(((1,), (0,)), ((), ())),
            preferred_element_type=jnp.float32)
        for ti in range(t):
            row3 = jnp.concatenate([
                ch,
                jnp.broadcast_to(pe[ti:ti + 1, :], (bs, n)),
                jnp.broadcast_to(me_b[ti:ti + 1, :], (bs, n)),
                zero,
            ], axis=-1)
            for q in range(reps):
                r0 = q * t * bs + ti * bs
                out_ref[bi, r0:r0 + bs, :] = row3


def _add_body(tok_ref, a_ref, out_ref):
    out_ref[...] = tok_ref[...] + a_ref[...]


def kernel(tokens, timestamps, channel_embed, pos_embed, month_table):
    b, h, w, t, bs, d = tokens.shape
    n = d // 4
    months = timestamps[:, :, 1].astype(jnp.int32)

    reps = 8
    period = t * bs
    blk_rows = reps * period

    a_exp = pl.pallas_call(
        _addend_body,
        out_shape=jax.ShapeDtypeStruct((b, blk_rows, d), jnp.float32),
    )(months, channel_embed, pos_embed, month_table)

    total_rows = b * h * w * t * bs
    nblk = total_rows // blk_rows
    blk_per_b = nblk // b
    tokens3 = tokens.reshape(nblk, blk_rows, d)

    out = pl.pallas_call(
        _add_body,
        grid=(nblk,),
        in_specs=[
            pl.BlockSpec((1, blk_rows, d), lambda i: (i, 0, 0)),
            pl.BlockSpec((1, blk_rows, d), lambda i: (i // blk_per_b, 0, 0)),
        ],
        out_specs=pl.BlockSpec((1, blk_rows, d), lambda i: (i, 0, 0)),
        out_shape=jax.ShapeDtypeStruct(tokens3.shape, tokens.dtype),
    )(tokens3, a_exp)
    return out.reshape(tokens.shape)


# speedup vs baseline: 1.8871x; 1.8871x over previous
import jax
import jax.numpy as jnp
from jax.experimental import pallas as pl
from jax.experimental.pallas import tpu as pltpu


def _addend_body(months_ref, ch_ref, pos_ref, mon_ref, out_ref):
    b, t, bs, d = out_ref.shape
    n = ch_ref.shape[1]
    ch = ch_ref[...]
    zero = jnp.zeros((bs, n), jnp.float32)
    for bi in range(b):
        for ti in range(t):
            m = months_ref[bi, ti]
            row_m = mon_ref[pl.ds(m, 1), :]
            row3 = jnp.concatenate([
                ch,
                jnp.broadcast_to(pos_ref[ti:ti + 1, :], (bs, n)),
                jnp.broadcast_to(row_m, (bs, n)),
                zero,
            ], axis=-1)
            out_ref[bi, ti] = row3


def _add_body(tok_ref, a_ref, out_ref):
    a = a_ref[...]
    out_ref[...] = tok_ref[...] + a[:, None, None]


def kernel(tokens, timestamps, channel_embed, pos_embed, month_table):
    b, h, w, t, bs, d = tokens.shape
    months = timestamps[:, :, 1].astype(jnp.int32)

    a_small = pl.pallas_call(
        _addend_body,
        in_specs=[
            pl.BlockSpec(memory_space=pltpu.SMEM),
            pl.BlockSpec(memory_space=pltpu.VMEM),
            pl.BlockSpec(memory_space=pltpu.VMEM),
            pl.BlockSpec(memory_space=pltpu.VMEM),
        ],
        out_shape=jax.ShapeDtypeStruct((b, t, bs, d), jnp.float32),
    )(months, channel_embed, pos_embed, month_table)

    H = 2
    out = pl.pallas_call(
        _add_body,
        grid=(b, h // H),
        in_specs=[
            pl.BlockSpec((1, H, w, t, bs, d), lambda i, j: (i, j, 0, 0, 0, 0)),
            pl.BlockSpec((1, t, bs, d), lambda i, j: (i, 0, 0, 0)),
        ],
        out_specs=pl.BlockSpec((1, H, w, t, bs, d), lambda i, j: (i, j, 0, 0, 0, 0)),
        out_shape=jax.ShapeDtypeStruct(tokens.shape, tokens.dtype),
    )(tokens, a_small)
    return out
